# baseline (speedup 1.0000x reference)
import jax
import jax.numpy as jnp
from jax.experimental import pallas as pl
from jax.experimental.pallas import tpu as pltpu

N_CHUNKS = 16
ROWS_PER_CHUNK = 64 // N_CHUNKS


def _copy_kernel(x_ref, o_ref, sems):
    copies = []
    for i in range(N_CHUNKS):
        sl = pl.ds(i * ROWS_PER_CHUNK, ROWS_PER_CHUNK)
        c = pltpu.make_async_copy(x_ref.at[sl], o_ref.at[sl], sems.at[i])
        c.start()
        copies.append(c)
    for c in copies:
        c.wait()


def kernel(samples, sample_rate):
    x = samples.reshape(64, 160000)
    out = pl.pallas_call(
        _copy_kernel,
        in_specs=[pl.BlockSpec(memory_space=pl.ANY)],
        out_specs=pl.BlockSpec(memory_space=pl.ANY),
        out_shape=jax.ShapeDtypeStruct(x.shape, x.dtype),
        scratch_shapes=[pltpu.SemaphoreType.DMA((N_CHUNKS,))],
    )(x)
    return out.reshape(samples.shape)


# speedup vs baseline: 12.3972x; 12.3972x over previous
import jax
import jax.numpy as jnp
from jax.experimental import pallas as pl
from jax.experimental.pallas import tpu as pltpu

GRID = 8
ROWS = 64 // GRID


def _copy_kernel(x_ref, o_ref):
    o_ref[...] = x_ref[...]


def kernel(samples, sample_rate):
    x = samples.reshape(64, 160000)
    out = pl.pallas_call(
        _copy_kernel,
        grid=(GRID,),
        in_specs=[pl.BlockSpec((ROWS, 160000), lambda i: (i, 0))],
        out_specs=pl.BlockSpec((ROWS, 160000), lambda i: (i, 0)),
        out_shape=jax.ShapeDtypeStruct(x.shape, x.dtype),
    )(x)
    return out.reshape(samples.shape)
